# Initial kernel scaffold; baseline (speedup 1.0000x reference)
import jax
import jax.numpy as jnp
from jax.experimental import pallas as pl


def kernel(x, weight):
    raise NotImplementedError("write your pallas kernel here")


# speedup vs baseline: 1.0493x; 1.0493x over previous
import jax
import jax.numpy as jnp
from jax.experimental import pallas as pl
from jax.experimental.pallas import tpu as pltpu

_EPS2 = 1e-12
_BM = 1024
_BN = 1024


def _euclid_block(x_ref, w_ref, o_ref):
    xb = x_ref[...]
    wb = w_ref[...]
    x2 = jnp.sum(xb * xb, axis=1, keepdims=True)
    w2 = jnp.sum(wb * wb, axis=0, keepdims=True)
    xw = jnp.dot(xb, wb, preferred_element_type=jnp.float32)
    d2 = jnp.maximum(x2 + w2 - 2.0 * xw, _EPS2)
    o_ref[...] = jnp.sqrt(d2)


def kernel(x, weight):
    b, k = x.shape
    _, o = weight.shape
    grid = (o // _BN, b // _BM)
    return pl.pallas_call(
        _euclid_block,
        out_shape=jax.ShapeDtypeStruct((b, o), jnp.float32),
        grid=grid,
        in_specs=[
            pl.BlockSpec((_BM, k), lambda j, i: (i, 0)),
            pl.BlockSpec((k, _BN), lambda j, i: (0, j)),
        ],
        out_specs=pl.BlockSpec((_BM, _BN), lambda j, i: (i, j)),
        compiler_params=pltpu.CompilerParams(
            dimension_semantics=("parallel", "arbitrary"),
        ),
        name="euclidean_fused",
    )(x, weight)


# speedup vs baseline: 1.1695x; 1.1146x over previous
import jax
import jax.numpy as jnp
from jax.experimental import pallas as pl
from jax.experimental.pallas import tpu as pltpu

_EPS2 = 1e-12
_BN = 1024


def _euclid_block(x_ref, w_ref, o_ref):
    xb = x_ref[...]
    wb = w_ref[...]
    x2 = jnp.sum(xb * xb, axis=1, keepdims=True)
    w2 = jnp.sum(wb * wb, axis=0, keepdims=True)
    xw = jnp.dot(xb, wb, preferred_element_type=jnp.float32)
    d2 = jnp.maximum(x2 + w2 - 2.0 * xw, _EPS2)
    o_ref[...] = jnp.sqrt(d2)


def kernel(x, weight):
    b, k = x.shape
    _, o = weight.shape
    grid = (o // _BN,)
    return pl.pallas_call(
        _euclid_block,
        out_shape=jax.ShapeDtypeStruct((b, o), jnp.float32),
        grid=grid,
        in_specs=[
            pl.BlockSpec((b, k), lambda j: (0, 0)),
            pl.BlockSpec((k, _BN), lambda j: (0, j)),
        ],
        out_specs=pl.BlockSpec((b, _BN), lambda j: (0, j)),
        compiler_params=pltpu.CompilerParams(
            dimension_semantics=("arbitrary",),
            vmem_limit_bytes=56 * 1024 * 1024,
        ),
        name="euclidean_fused",
    )(x, weight)


# speedup vs baseline: 1.5162x; 1.2965x over previous
import jax
import jax.numpy as jnp
from jax.experimental import pallas as pl
from jax.experimental.pallas import tpu as pltpu

_EPS2 = 1e-12
_BN = 1024


def _euclid_block(x_ref, w_ref, o_ref, xs_ref, x2_ref):
    @pl.when(pl.program_id(0) == 0)
    def _():
        xb = x_ref[...]
        x2_ref[...] = jnp.sum(xb * xb, axis=1, keepdims=True)
        xs_ref[...] = xb * -2.0
    wb = w_ref[...]
    w2 = jnp.sum(wb * wb, axis=0, keepdims=True)
    xw = jnp.dot(xs_ref[...], wb, preferred_element_type=jnp.float32)
    d2 = jnp.maximum(xw + x2_ref[...] + w2, _EPS2)
    o_ref[...] = d2 * jax.lax.rsqrt(d2)


def kernel(x, weight):
    b, k = x.shape
    _, o = weight.shape
    grid = (o // _BN,)
    return pl.pallas_call(
        _euclid_block,
        out_shape=jax.ShapeDtypeStruct((b, o), jnp.float32),
        grid=grid,
        in_specs=[
            pl.BlockSpec((b, k), lambda j: (0, 0)),
            pl.BlockSpec((k, _BN), lambda j: (0, j)),
        ],
        out_specs=pl.BlockSpec((b, _BN), lambda j: (0, j)),
        scratch_shapes=[
            pltpu.VMEM((b, k), jnp.float32),
            pltpu.VMEM((b, 1), jnp.float32),
        ],
        compiler_params=pltpu.CompilerParams(
            dimension_semantics=("arbitrary",),
            vmem_limit_bytes=56 * 1024 * 1024,
        ),
        name="euclidean_fused",
    )(x, weight)


# speedup vs baseline: 1.6271x; 1.0731x over previous
import jax
import jax.numpy as jnp
from jax.experimental import pallas as pl
from jax.experimental.pallas import tpu as pltpu

_EPS2 = 1e-12
_BN = 1024
_BMC = 256


def _euclid_block(x_ref, w_ref, o_ref, xs_ref, x2_ref):
    @pl.when(pl.program_id(0) == 0)
    def _():
        xb = x_ref[...]
        x2_ref[...] = jnp.sum(xb * xb, axis=1, keepdims=True)
        xs_ref[...] = xb * -2.0
    wb = w_ref[...]
    w2 = jnp.sum(wb * wb, axis=0, keepdims=True)
    b = x_ref.shape[0]
    for i in range(0, b, _BMC):
        sl = pl.ds(i, _BMC)
        xw = jnp.dot(xs_ref[sl, :], wb, preferred_element_type=jnp.float32)
        d2 = jnp.maximum(xw + x2_ref[sl, :] + w2, _EPS2)
        o_ref[sl, :] = d2 * jax.lax.rsqrt(d2)


def kernel(x, weight):
    b, k = x.shape
    _, o = weight.shape
    grid = (o // _BN,)
    return pl.pallas_call(
        _euclid_block,
        out_shape=jax.ShapeDtypeStruct((b, o), jnp.float32),
        grid=grid,
        in_specs=[
            pl.BlockSpec((b, k), lambda j: (0, 0)),
            pl.BlockSpec((k, _BN), lambda j: (0, j)),
        ],
        out_specs=pl.BlockSpec((b, _BN), lambda j: (0, j)),
        scratch_shapes=[
            pltpu.VMEM((b, k), jnp.float32),
            pltpu.VMEM((b, 1), jnp.float32),
        ],
        compiler_params=pltpu.CompilerParams(
            dimension_semantics=("arbitrary",),
            vmem_limit_bytes=56 * 1024 * 1024,
        ),
        name="euclidean_fused",
    )(x, weight)
